# baseline (speedup 1.0000x reference)
import functools

import jax
import jax.numpy as jnp
from jax import lax
from jax.experimental import pallas as pl
from jax.experimental.pallas import tpu as pltpu
from jax.experimental.pallas import tpu_sc as plsc

N_NODES = 100000
EMB_DIM = 64
D_FEAT = 128
OUT_DIM = EMB_DIM + D_FEAT
GW = 2 * EMB_DIM

NC = 2
NS = 16
NW = NC * NS

G = 80
NG = 5
S = G * NG
NBLK = N_NODES // S


def _gather_body(ids_h, table2, gout, idx_v, gbuf_v, gsem):
    wid = lax.axis_index("s") * NC + lax.axis_index("c")
    nblk_w = (NBLK - 1 - wid) // NW + 1

    def blk(i, _):
        k = wid + i * NW
        base = k * S
        pltpu.sync_copy(ids_h.at[pl.ds(base, S)], idx_v)
        handles = []
        for j in range(NG):
            h = pltpu.async_copy(
                table2.at[idx_v.at[pl.ds(j * G, G)]],
                gbuf_v.at[pl.ds(j * G, G), :],
                gsem,
            )
            handles.append(h)
        for h in handles:
            h.wait()
        pltpu.sync_copy(gbuf_v, gout.at[pl.ds(base, S), :])
        return 0

    lax.fori_loop(0, nblk_w, blk, 0)


def _sc_gather(ids_q, table2):
    mesh = plsc.VectorSubcoreMesh(core_axis_name="c", subcore_axis_name="s")
    return pl.kernel(
        _gather_body,
        mesh=mesh,
        out_type=jax.ShapeDtypeStruct((N_NODES, GW), jnp.float32),
        scratch_types=[
            pltpu.VMEM((S,), jnp.int32),
            pltpu.VMEM((S, GW), jnp.float32),
            pltpu.SemaphoreType.DMA,
        ],
    )(ids_q, table2)


BR = 2000


def _concat_body(g_ref, rem_ref, feats_ref, out_ref):
    rem = rem_ref[...]
    lo = g_ref[:, 0:EMB_DIM]
    hi = g_ref[:, EMB_DIM:GW]
    out_ref[:, 0:EMB_DIM] = jnp.where(rem == 1, hi, lo)
    out_ref[:, EMB_DIM:OUT_DIM] = feats_ref[...]


def _tc_concat(gath, rem, feats):
    return pl.pallas_call(
        _concat_body,
        grid=(N_NODES // BR,),
        in_specs=[
            pl.BlockSpec((BR, GW), lambda i: (i, 0)),
            pl.BlockSpec((BR, 1), lambda i: (i, 0)),
            pl.BlockSpec((BR, D_FEAT), lambda i: (i, 0)),
        ],
        out_specs=pl.BlockSpec((BR, OUT_DIM), lambda i: (i, 0)),
        out_shape=jax.ShapeDtypeStruct((N_NODES, OUT_DIM), jnp.float32),
    )(gath, rem, feats)


@jax.jit
def _run(ids, feats, table):
    ids = ids.astype(jnp.int32)
    table2 = table.reshape(-1, GW)
    gath = _sc_gather(ids // 2, table2)
    rem = (ids % 2).reshape(-1, 1)
    return _tc_concat(gath, rem, feats)


def kernel(ids, feats, table):
    return _run(ids, feats, table)


# speedup vs baseline: 1.0030x; 1.0030x over previous
import functools

import jax
import jax.numpy as jnp
from jax import lax
from jax.experimental import pallas as pl
from jax.experimental.pallas import tpu as pltpu
from jax.experimental.pallas import tpu_sc as plsc

N_NODES = 100000
EMB_DIM = 64
D_FEAT = 128
OUT_DIM = EMB_DIM + D_FEAT
GW = 2 * EMB_DIM

NC = 2
NS = 16
NW = NC * NS

G = 80
NG = 5
S = G * NG
NBLK = N_NODES // S


NBLK_W = (NBLK + NW - 1) // NW


def _gather_body(ids_h, table2, gout,
                 idx0, idx1, gb0, gb1,
                 isem0, isem1, gsem0, gsem1, wsem0, wsem1):
    wid = lax.axis_index("s") * NC + lax.axis_index("c")
    idx = (idx0, idx1)
    gb = (gb0, gb1)
    isem = (isem0, isem1)
    wsem = (wsem0, wsem1)
    gsem = (gsem0, gsem1)

    ih = [None]
    wbh = {}

    def base_of(i):
        k = wid + i * NW
        k = jnp.where(k < NBLK, k, k - NW)
        return k * S

    for i in range(NBLK_W):
        p = i % 2
        base = base_of(i)
        if i >= 2:
            wbh[i - 2].wait()
        if i == 0:
            pltpu.sync_copy(ids_h.at[pl.ds(base, S)], idx[p])
        else:
            ih[0].wait()
        ghs = []
        for j in range(NG):
            ghs.append(pltpu.async_copy(
                table2.at[idx[p].at[pl.ds(j * G, G)]],
                gb[p].at[pl.ds(j * G, G), :],
                gsem[p],
            ))
        if i + 1 < NBLK_W:
            ih[0] = pltpu.async_copy(
                ids_h.at[pl.ds(base_of(i + 1), S)],
                idx[(i + 1) % 2], isem[(i + 1) % 2])
        for h in ghs:
            h.wait()
        wbh[i] = pltpu.async_copy(
            gb[p], gout.at[pl.ds(base, S), :], wsem[p])

    wbh[NBLK_W - 2].wait()
    wbh[NBLK_W - 1].wait()


def _sc_gather(ids_q, table2):
    mesh = plsc.VectorSubcoreMesh(core_axis_name="c", subcore_axis_name="s")
    return pl.kernel(
        _gather_body,
        mesh=mesh,
        out_type=jax.ShapeDtypeStruct((N_NODES, GW), jnp.float32),
        scratch_types=[
            pltpu.VMEM((S,), jnp.int32),
            pltpu.VMEM((S,), jnp.int32),
            pltpu.VMEM((S, GW), jnp.float32),
            pltpu.VMEM((S, GW), jnp.float32),
            pltpu.SemaphoreType.DMA,
            pltpu.SemaphoreType.DMA,
            pltpu.SemaphoreType.DMA,
            pltpu.SemaphoreType.DMA,
            pltpu.SemaphoreType.DMA,
            pltpu.SemaphoreType.DMA,
        ],
    )(ids_q, table2)


BR = 2000


def _concat_body(g_ref, rem_ref, feats_ref, out_ref):
    rem = rem_ref[...]
    lo = g_ref[:, 0:EMB_DIM]
    hi = g_ref[:, EMB_DIM:GW]
    out_ref[:, 0:EMB_DIM] = jnp.where(rem == 1, hi, lo)
    out_ref[:, EMB_DIM:OUT_DIM] = feats_ref[...]


def _tc_concat(gath, rem, feats):
    return pl.pallas_call(
        _concat_body,
        grid=(N_NODES // BR,),
        in_specs=[
            pl.BlockSpec((BR, GW), lambda i: (i, 0)),
            pl.BlockSpec((BR, 1), lambda i: (i, 0)),
            pl.BlockSpec((BR, D_FEAT), lambda i: (i, 0)),
        ],
        out_specs=pl.BlockSpec((BR, OUT_DIM), lambda i: (i, 0)),
        out_shape=jax.ShapeDtypeStruct((N_NODES, OUT_DIM), jnp.float32),
    )(gath, rem, feats)


@jax.jit
def _run(ids, feats, table):
    ids = ids.astype(jnp.int32)
    table2 = table.reshape(-1, GW)
    gath = _sc_gather(ids // 2, table2)
    rem = (ids % 2).reshape(-1, 1)
    return _tc_concat(gath, rem, feats)


def kernel(ids, feats, table):
    return _run(ids, feats, table)
